# baseline (speedup 1.0000x reference)
import numpy as np
import jax
import jax.numpy as jnp
from jax import lax
from jax.experimental import pallas as pl
from jax.experimental.pallas import tpu as pltpu
from jax.experimental.pallas import tpu_sc as plsc

MAXLEN_ = 2048
D_MODEL_ = 768
BATCH_ = 4
LANES_ = 16

NW_ = 32
ROWS_ = BATCH_ * MAXLEN_
POS_W_ = MAXLEN_ // NW_
CHUNK_ = 32
HALF_ = POS_W_ // CHUNK_
NCHUNK_ = BATCH_ * HALF_
VECS_ = D_MODEL_ // LANES_


def _positional_encoding(maxlen, d_model):
    pos = np.arange(maxlen, dtype=np.float32)[:, None]
    i = np.arange(d_model, dtype=np.float32)[None, :]
    angle_rates = 1.0 / np.power(10000.0, (2.0 * np.floor(i / 2.0)) / np.float32(d_model))
    angle_rads = pos * angle_rates
    pe = np.zeros((maxlen, d_model), dtype=np.float32)
    pe[:, 0::2] = np.sin(angle_rads[:, 0::2])
    pe[:, 1::2] = np.cos(angle_rads[:, 1::2])
    return pe


NBUF_ = 4
AHEAD_ = 2


def _chunk_idx(k):
    return k // BATCH_, k % BATCH_


def _emb_body(table_hbm, x_hbm, pe_hbm, out_hbm,
              idx_v, b0, b1, b2, b3, pe_v,
              g0, g1, g2, g3, o0, o1, o2, o3):
    wid = lax.axis_index("s") * 2 + lax.axis_index("c")
    pbase = wid * POS_W_
    bufs = (b0, b1, b2, b3)
    gsems = (g0, g1, g2, g3)
    osems = (o0, o1, o2, o3)

    gcopy = [None] * NBUF_
    ocopy = [None] * NBUF_

    def gather(k):
        h, b = _chunk_idx(k)
        return pltpu.async_copy(
            table_hbm.at[idx_v.at[b, pl.ds(h * CHUNK_, CHUNK_)]],
            bufs[k % NBUF_], gsems[k % NBUF_])

    pltpu.sync_copy(x_hbm.at[0, pl.ds(pbase, POS_W_)], idx_v.at[0])
    gcopy[0] = gather(0)
    for b in range(1, BATCH_):
        pltpu.sync_copy(x_hbm.at[b, pl.ds(pbase, POS_W_)], idx_v.at[b])
    gcopy[1] = gather(1)
    pltpu.sync_copy(pe_hbm.at[pl.ds(pbase, POS_W_)], pe_v)

    for k in range(NCHUNK_):
        h, b = _chunk_idx(k)
        buf = bufs[k % NBUF_]
        gcopy[k % NBUF_].wait()
        if k + AHEAD_ < NCHUNK_:
            nb = (k + AHEAD_) % NBUF_
            if ocopy[nb] is not None:
                for oc in ocopy[nb]:
                    oc.wait()
                ocopy[nb] = None
            gcopy[nb] = gather(k + AHEAD_)

        def add_rows(base_r, nrows):
            def body(i, carry):
                r = base_r + i * 2
                for rr in (r, r + 1):
                    for j in range(VECS_ // 2):
                        vi = pe_v[h * CHUNK_ + rr, pl.ds(j * LANES_, LANES_)]
                        shift16 = jnp.full((LANES_,), 16, jnp.int32)
                        maskhi = jnp.full((LANES_,), -65536, jnp.int32)
                        lo = lax.bitcast_convert_type(
                            lax.shift_left(vi, shift16), jnp.float32)
                        hi = lax.bitcast_convert_type(
                            lax.bitwise_and(vi, maskhi), jnp.float32)
                        plsc.addupdate(
                            buf.at[rr, pl.ds(j * 2 * LANES_, LANES_)], lo)
                        plsc.addupdate(
                            buf.at[rr, pl.ds(j * 2 * LANES_ + LANES_, LANES_)], hi)
                return carry
            lax.fori_loop(0, nrows // 2, body, 0)

        obase = b * MAXLEN_ + pbase + h * CHUNK_
        if k < NCHUNK_ - 1:
            add_rows(0, CHUNK_)
            ocopy[k % NBUF_] = (pltpu.async_copy(
                buf, out_hbm.at[pl.ds(obase, CHUNK_)], osems[k % NBUF_]),)
        else:
            add_rows(0, CHUNK_ // 2)
            oc1 = pltpu.async_copy(buf.at[pl.ds(0, CHUNK_ // 2)],
                                   out_hbm.at[pl.ds(obase, CHUNK_ // 2)],
                                   osems[k % NBUF_])
            add_rows(CHUNK_ // 2, CHUNK_ // 2)
            oc2 = pltpu.async_copy(
                buf.at[pl.ds(CHUNK_ // 2, CHUNK_ // 2)],
                out_hbm.at[pl.ds(obase + CHUNK_ // 2, CHUNK_ // 2)],
                osems[k % NBUF_])
            ocopy[k % NBUF_] = (oc1, oc2)

    for pair in ocopy:
        if pair is not None:
            for oc in pair:
                oc.wait()


def kernel(x, token_emb_table):
    pe_np = _positional_encoding(MAXLEN_, D_MODEL_)
    pe_bf = jnp.asarray(pe_np).astype(jnp.bfloat16)
    pe_pairs = (pe_bf.reshape(MAXLEN_, VECS_ // 2, 2, LANES_)
                     .transpose(0, 1, 3, 2))
    pe = lax.bitcast_convert_type(pe_pairs, jnp.int32).reshape(
        MAXLEN_, D_MODEL_ // 2)
    mesh = plsc.VectorSubcoreMesh(core_axis_name="c", subcore_axis_name="s")
    out = pl.kernel(
        _emb_body,
        out_type=jax.ShapeDtypeStruct((ROWS_, D_MODEL_), jnp.float32),
        mesh=mesh,
        scratch_types=(
            [pltpu.VMEM((BATCH_, POS_W_), jnp.int32)]
            + [pltpu.VMEM((CHUNK_, D_MODEL_), jnp.float32)] * NBUF_
            + [pltpu.VMEM((POS_W_, D_MODEL_ // 2), jnp.int32)]
            + [pltpu.SemaphoreType.DMA] * (2 * NBUF_)
        ),
    )(token_emb_table, x.astype(jnp.int32), pe)
    return out.reshape(BATCH_, MAXLEN_, D_MODEL_)


# speedup vs baseline: 1.3099x; 1.3099x over previous
import numpy as np
import jax
import jax.numpy as jnp
from jax import lax
from jax.experimental import pallas as pl
from jax.experimental.pallas import tpu as pltpu
from jax.experimental.pallas import tpu_sc as plsc

MAXLEN_ = 2048
D_MODEL_ = 768
BATCH_ = 4
LANES_ = 16

NW_ = 32
ROWS_ = BATCH_ * MAXLEN_
POS_W_ = MAXLEN_ // NW_
CHUNK_ = 32
HALF_ = POS_W_ // CHUNK_
NCHUNK_ = BATCH_ * HALF_
VECS_ = D_MODEL_ // LANES_
NBUF_ = 4
AHEAD_ = 2


def _positional_encoding(maxlen, d_model):
    pos = np.arange(maxlen, dtype=np.float32)[:, None]
    i = np.arange(d_model, dtype=np.float32)[None, :]
    angle_rates = 1.0 / np.power(10000.0, (2.0 * np.floor(i / 2.0)) / np.float32(d_model))
    angle_rads = pos * angle_rates
    pe = np.zeros((maxlen, d_model), dtype=np.float32)
    pe[:, 0::2] = np.sin(angle_rads[:, 0::2])
    pe[:, 1::2] = np.cos(angle_rads[:, 1::2])
    return pe


def _chunk_idx(k):
    return k // BATCH_, k % BATCH_


def _emb_body(table_hbm, x_hbm, pe_hbm, out_hbm,
              idx_v, b0, b1, b2, b3, pe_v,
              g0, g1, g2, g3, o0, o1, o2, o3):
    wid = lax.axis_index("s") * 2 + lax.axis_index("c")
    pbase = wid * POS_W_
    bufs = (b0, b1, b2, b3)
    gsems = (g0, g1, g2, g3)
    osems = (o0, o1, o2, o3)

    gcopy = [None] * NBUF_
    ocopy = [None] * NBUF_

    def gather(k):
        h, b = _chunk_idx(k)
        return pltpu.async_copy(
            table_hbm.at[idx_v.at[b, pl.ds(h * CHUNK_, CHUNK_)]],
            bufs[k % NBUF_], gsems[k % NBUF_])

    pltpu.sync_copy(x_hbm.at[0, pl.ds(pbase, POS_W_)], idx_v.at[0])
    gcopy[0] = gather(0)
    for b in range(1, BATCH_):
        pltpu.sync_copy(x_hbm.at[b, pl.ds(pbase, POS_W_)], idx_v.at[b])
    gcopy[1] = gather(1)
    pltpu.sync_copy(pe_hbm.at[pl.ds(pbase, CHUNK_)], pe_v)

    for k in range(NCHUNK_):
        h, b = _chunk_idx(k)
        if k == BATCH_:
            pltpu.sync_copy(
                pe_hbm.at[pl.ds(pbase + CHUNK_, CHUNK_)], pe_v)
        buf = bufs[k % NBUF_]
        gcopy[k % NBUF_].wait()
        if k + AHEAD_ < NCHUNK_:
            nb = (k + AHEAD_) % NBUF_
            if ocopy[nb] is not None:
                ocopy[nb].wait()
                ocopy[nb] = None
            gcopy[nb] = gather(k + AHEAD_)

        def add_rows(i, carry):
            r = i * 2
            for rr in (r, r + 1):
                for j in range(VECS_):
                    sl = pl.ds(j * LANES_, LANES_)
                    plsc.addupdate(buf.at[rr, sl], pe_v[rr, sl])
            return carry

        lax.fori_loop(0, CHUNK_ // 2, add_rows, 0)
        obase = b * MAXLEN_ + pbase + h * CHUNK_
        ocopy[k % NBUF_] = pltpu.async_copy(
            buf, out_hbm.at[pl.ds(obase, CHUNK_)], osems[k % NBUF_])

    for oc in ocopy:
        if oc is not None:
            oc.wait()


def kernel(x, token_emb_table):
    pe = jnp.asarray(_positional_encoding(MAXLEN_, D_MODEL_))
    mesh = plsc.VectorSubcoreMesh(core_axis_name="c", subcore_axis_name="s")
    out = pl.kernel(
        _emb_body,
        out_type=jax.ShapeDtypeStruct((ROWS_, D_MODEL_), jnp.float32),
        mesh=mesh,
        scratch_types=(
            [pltpu.VMEM((BATCH_, POS_W_), jnp.int32)]
            + [pltpu.VMEM((CHUNK_, D_MODEL_), jnp.float32)] * NBUF_
            + [pltpu.VMEM((CHUNK_, D_MODEL_), jnp.float32)]
            + [pltpu.SemaphoreType.DMA] * (2 * NBUF_)
        ),
    )(token_emb_table, x.astype(jnp.int32), pe)
    return out.reshape(BATCH_, MAXLEN_, D_MODEL_)
